# baseline (speedup 1.0000x reference)
import functools

import jax
import jax.numpy as jnp
from jax import lax
from jax.experimental import pallas as pl
from jax.experimental.pallas import tpu as pltpu
from jax.experimental.pallas import tpu_sc as plsc

N_NODES = 10000
N_EDGES = 320000
D = 128
G = 64

NC, NS, L = 2, 16, 16
NW = NC * NS
NPAD = 10240
NPT = NPAD // NW
EPT = N_EDGES // NW
GC = 80
GD = 80

_mesh = plsc.VectorSubcoreMesh(core_axis_name="c", subcore_axis_name="s")


def _wid():
    return lax.axis_index("s") * NC + lax.axis_index("c")


def _split_body(ei_ref, src_ref, dst_ref):
    ei = ei_ref[...]
    src_ref[...] = ei[0]
    dst_ref[...] = ei[1]


def _split(edge_index):
    return pl.pallas_call(
        _split_body,
        out_shape=[jax.ShapeDtypeStruct((N_EDGES,), jnp.int32),
                   jax.ShapeDtypeStruct((N_EDGES,), jnp.int32)],
    )(edge_index)


HALF = NPAD // NC
EPS = N_EDGES // NS


@functools.partial(
    pl.kernel,
    out_type=jax.ShapeDtypeStruct((NPAD, D), jnp.float32),
    mesh=_mesh,
    compiler_params=pltpu.CompilerParams(needs_layout_passes=False),
    scratch_types=[
        pltpu.VMEM((EPS,), jnp.int32),
        pltpu.VMEM((HALF,), jnp.int32),
        pltpu.VMEM((NS * NPT,), jnp.int32),
        pltpu.VMEM((NPT,), jnp.int32),
        pltpu.VMEM((NPT, D), jnp.float32),
        pltpu.VMEM((EPS,), jnp.int32),
        pltpu.VMEM_SHARED((NS * HALF,), jnp.int32),
        pltpu.VMEM_SHARED((N_EDGES,), jnp.int32),
        pltpu.SemaphoreType.DMA,
        pltpu.SemaphoreType.DMA,
    ],
)
def _winners_gather(dst_hbm, src_hbm, x_hbm, xs_hbm, dst_v, win_v, tabs_v,
                    sidx_v, xrows_v, src_v, sh_v, src_sh, sem_a, sem_b):
    cid = lax.axis_index("c")
    sid = lax.axis_index("s")
    lo = cid * HALF
    base_e = sid * EPS
    pltpu.async_copy(src_hbm.at[pl.ds(base_e, EPS)], src_v, sem_b)
    pltpu.sync_copy(dst_hbm.at[pl.ds(base_e, EPS)], dst_v)

    lane = lax.iota(jnp.int32, L)
    neg1 = jnp.full((L,), -1, jnp.int32)

    def init_body(i, _):
        win_v[pl.ds(i * L, L)] = neg1
        return 0

    lax.fori_loop(0, HALF // L, init_body, 0)

    def body(c, _):
        d16 = dst_v[pl.ds(c * L, L)]
        ld = d16 - lo
        valid = (ld >= 0) & (ld < HALF)
        ldc = jnp.minimum(jnp.maximum(ld, 0), HALF - 1)
        e16 = base_e + c * L + lane
        plsc.store_scatter(win_v, [ldc], e16, mask=valid)
        return 0

    lax.fori_loop(0, EPS // L, body, 0)
    pltpu.sync_copy(win_v, sh_v.at[pl.ds(sid * HALF, HALF)])
    pltpu.make_async_copy(src_hbm.at[pl.ds(0, EPS)], src_v, sem_b).wait()
    pltpu.sync_copy(src_v, src_sh.at[pl.ds(base_e, EPS)])
    plsc.subcore_barrier()

    for t in range(NS):
        pltpu.async_copy(sh_v.at[pl.ds(t * HALF + sid * NPT, NPT)],
                         tabs_v.at[pl.ds(t * NPT, NPT)], sem_a)
    for t in range(NS):
        pltpu.make_async_copy(sh_v.at[pl.ds(sid * NPT, NPT)],
                              tabs_v.at[pl.ds(0, NPT)], sem_a).wait()

    def merge_body(i, _):
        m = tabs_v[pl.ds(i * L, L)]
        for t in range(1, NS):
            m = jnp.maximum(m, tabs_v[pl.ds(t * NPT + i * L, L)])
        tabs_v[pl.ds(i * L, L)] = jnp.maximum(m, 0)
        tabs_v[pl.ds(NPT + i * L, L)] = (m >> 31)
        return 0

    lax.fori_loop(0, NPT // L, merge_body, 0)

    for c in range(NPT // GC):
        pltpu.async_copy(src_sh.at[tabs_v.at[pl.ds(c * GC, GC)]],
                         sidx_v.at[pl.ds(c * GC, GC)], sem_a)
    for c in range(NPT // GC):
        pltpu.make_async_copy(src_sh.at[tabs_v.at[pl.ds(0, GC)]],
                              sidx_v.at[pl.ds(0, GC)], sem_a).wait()

    def fix_body(i, _):
        miss = tabs_v[pl.ds(NPT + i * L, L)]
        s16 = sidx_v[pl.ds(i * L, L)]
        own = lo + sid * NPT + i * L + lane
        sidx_v[pl.ds(i * L, L)] = jnp.where(miss != 0, own, s16)
        return 0

    lax.fori_loop(0, NPT // L, fix_body, 0)

    for c in range(NPT // GC):
        pltpu.async_copy(x_hbm.at[sidx_v.at[pl.ds(c * GC, GC)]],
                         xrows_v.at[pl.ds(c * GC, GC)], sem_b)
    for c in range(NPT // GC):
        pltpu.make_async_copy(x_hbm.at[sidx_v.at[pl.ds(0, GC)]],
                              xrows_v.at[pl.ds(0, GC)], sem_b).wait()
    pltpu.sync_copy(xrows_v, xs_hbm.at[pl.ds(lo + sid * NPT, NPT)])


def _dense_body(x_ref, xs_ref, wq, bq, wk, bk, wv, bv, wr, br, gma, bta, ws,
                bs, wsc, out_ref):
    xs = xs_ref[...]
    xx = x_ref[...]
    q = jnp.dot(xs, wq[...], preferred_element_type=jnp.float32) + bq[...]
    k = jnp.dot(xx, wk[...], preferred_element_type=jnp.float32) + bk[...]
    v = jnp.dot(xx, wv[...], preferred_element_type=jnp.float32) + bv[...]
    scores = (q * k) * (1.0 / 8.0)
    scores = scores - jnp.max(scores, axis=-1, keepdims=True)
    ex = jnp.exp(scores)
    attn = ex / jnp.sum(ex, axis=-1, keepdims=True)
    attn_out = attn * v
    mu = jnp.mean(attn_out, axis=-1, keepdims=True)
    ctr = attn_out - mu
    var = jnp.mean(ctr * ctr, axis=-1, keepdims=True)
    xn = ctr * lax.rsqrt(var + 1e-5) * gma[...] + bta[...]
    h = attn_out + jnp.maximum(
        jnp.dot(xn, wr[...], preferred_element_type=jnp.float32) + br[...], 0.0)
    gate = jnp.dot(h, ws[...], preferred_element_type=jnp.float32) + bs[...] \
        + attn_out
    out_ref[...] = xs - xx + jnp.dot(gate, wsc[...],
                                     preferred_element_type=jnp.float32)


def _dense(x_pad, xs, hp, w_scale):
    blk = 640
    grid = NPAD // blk

    def row_spec(dim):
        return pl.BlockSpec((blk, dim), lambda i: (i, 0))

    def full_spec(a):
        return pl.BlockSpec(a.shape, lambda i: (0,) * a.ndim)

    weights = [hp['Wq'], hp['bq'].reshape(1, G), hp['Wk'], hp['bk'].reshape(1, G),
               hp['Wv'], hp['bv'].reshape(1, G), hp['res'][0]['W'],
               hp['res'][0]['b'].reshape(1, G), hp['res'][0]['gamma'].reshape(1, G),
               hp['res'][0]['beta'].reshape(1, G), hp['Ws'],
               hp['bs'].reshape(1, G), w_scale]
    return pl.pallas_call(
        _dense_body,
        out_shape=jax.ShapeDtypeStruct((NPAD, D), jnp.float32),
        grid=(grid,),
        in_specs=[row_spec(D), row_spec(D)] + [full_spec(w) for w in weights],
        out_specs=row_spec(D),
    )(x_pad, xs, *weights)


@functools.partial(
    pl.kernel,
    out_type=jax.ShapeDtypeStruct((N_EDGES, D), jnp.float32),
    mesh=_mesh,
    scratch_types=[
        pltpu.VMEM((EPT,), jnp.int32),
        pltpu.VMEM((GD, D), jnp.float32),
        pltpu.VMEM((GD, D), jnp.float32),
        pltpu.VMEM_SHARED((NPAD, D), jnp.float32),
        pltpu.SemaphoreType.DMA,
        pltpu.SemaphoreType.DMA,
        pltpu.SemaphoreType.DMA,
        pltpu.SemaphoreType.DMA,
        pltpu.SemaphoreType.DMA,
    ],
)
def _edge_gather(h_hbm, dst_hbm, out_hbm, dst_v, rows_a, rows_b, h_sh,
                 sem_l, sem_ga, sem_gb, sem_sa, sem_sb):
    wid = _wid()
    sid = lax.axis_index("s")
    base_e = wid * EPT
    rows_per_sub = NPAD // NS
    pltpu.async_copy(dst_hbm.at[pl.ds(base_e, EPT)], dst_v, sem_l)
    pltpu.sync_copy(h_hbm.at[pl.ds(sid * rows_per_sub, rows_per_sub)],
                    h_sh.at[pl.ds(sid * rows_per_sub, rows_per_sub)])
    pltpu.make_async_copy(dst_hbm.at[pl.ds(base_e, EPT)], dst_v, sem_l).wait()
    plsc.subcore_barrier()

    def g_start(c, buf, sem):
        pltpu.async_copy(h_sh.at[dst_v.at[pl.ds(c * GD, GD)]], buf, sem)

    def g_wait(buf, sem):
        pltpu.make_async_copy(h_sh.at[dst_v.at[pl.ds(0, GD)]], buf, sem).wait()

    def s_start(c, buf, sem):
        pltpu.async_copy(buf, out_hbm.at[pl.ds(base_e + c * GD, GD)], sem)

    def s_wait(buf, sem):
        pltpu.make_async_copy(buf, out_hbm.at[pl.ds(base_e, GD)], sem).wait()

    g_start(0, rows_a, sem_ga)

    def body(i, _):
        c = 2 * i

        @pl.when(i > 0)
        def _():
            s_wait(rows_b, sem_sb)

        g_start(c + 1, rows_b, sem_gb)
        g_wait(rows_a, sem_ga)
        s_start(c, rows_a, sem_sa)
        s_wait(rows_a, sem_sa)
        g_start(c + 2, rows_a, sem_ga)
        g_wait(rows_b, sem_gb)
        s_start(c + 1, rows_b, sem_sb)
        return 0

    n_pairs = (EPT // GD) // 2
    lax.fori_loop(0, n_pairs, body, 0)
    s_wait(rows_b, sem_sb)
    g_wait(rows_a, sem_ga)
    s_start(EPT // GD - 1, rows_a, sem_sa)
    s_wait(rows_a, sem_sa)


def kernel(x, edge_index, params):
    src, dst = _split(edge_index)
    hp = params['heads'][0]
    x_pad = jnp.concatenate(
        [x, jnp.zeros((NPAD - N_NODES, D), jnp.float32)], axis=0)

    xs = _winners_gather(dst, src, x_pad)
    h_node = _dense(x_pad, xs, hp, params['W_scale'])
    return _edge_gather(h_node, dst)


# speedup vs baseline: 1.0666x; 1.0666x over previous
import functools

import jax
import jax.numpy as jnp
from jax import lax
from jax.experimental import pallas as pl
from jax.experimental.pallas import tpu as pltpu
from jax.experimental.pallas import tpu_sc as plsc

N_NODES = 10000
N_EDGES = 320000
D = 128
G = 64

NC, NS, L = 2, 16, 16
NW = NC * NS
NPAD = 10240
NPT = NPAD // NW
EPT = N_EDGES // NW
GC = 80
GD = 80

_mesh = plsc.VectorSubcoreMesh(core_axis_name="c", subcore_axis_name="s")


def _wid():
    return lax.axis_index("s") * NC + lax.axis_index("c")


def _split_body(ei_ref, src_ref, dst_ref):
    ei = ei_ref[...]
    src_ref[...] = ei[0]
    dst_ref[...] = ei[1]


def _split(edge_index):
    return pl.pallas_call(
        _split_body,
        out_shape=[jax.ShapeDtypeStruct((N_EDGES,), jnp.int32),
                   jax.ShapeDtypeStruct((N_EDGES,), jnp.int32)],
    )(edge_index)


HALF = NPAD // NC
EPS = N_EDGES // NS


@functools.partial(
    pl.kernel,
    out_type=jax.ShapeDtypeStruct((NPAD, D), jnp.float32),
    mesh=_mesh,
    compiler_params=pltpu.CompilerParams(needs_layout_passes=False),
    scratch_types=[
        pltpu.VMEM((EPS,), jnp.int32),
        pltpu.VMEM((HALF,), jnp.int32),
        pltpu.VMEM((NS * NPT,), jnp.int32),
        pltpu.VMEM((NPT,), jnp.int32),
        pltpu.VMEM((NPT, D), jnp.float32),
        pltpu.VMEM((EPS,), jnp.int32),
        pltpu.VMEM_SHARED((NS * HALF,), jnp.int32),
        pltpu.VMEM_SHARED((N_EDGES,), jnp.int32),
        pltpu.SemaphoreType.DMA,
        pltpu.SemaphoreType.DMA,
    ],
)
def _winners_gather(dst_hbm, src_hbm, x_hbm, xs_hbm, dst_v, win_v, tabs_v,
                    sidx_v, xrows_v, src_v, sh_v, src_sh, sem_a, sem_b):
    cid = lax.axis_index("c")
    sid = lax.axis_index("s")
    lo = cid * HALF
    base_e = sid * EPS
    pltpu.async_copy(src_hbm.at[pl.ds(base_e, EPS)], src_v, sem_b)
    pltpu.sync_copy(dst_hbm.at[pl.ds(base_e, EPS)], dst_v)

    lane = lax.iota(jnp.int32, L)
    neg1 = jnp.full((L,), -1, jnp.int32)

    def init_body(i, _):
        win_v[pl.ds(i * L, L)] = neg1
        return 0

    lax.fori_loop(0, HALF // L, init_body, 0)

    def body(c, _):
        d16 = dst_v[pl.ds(c * L, L)]
        ld = d16 - lo
        valid = (ld >= 0) & (ld < HALF)
        ldc = jnp.minimum(jnp.maximum(ld, 0), HALF - 1)
        e16 = base_e + c * L + lane
        plsc.store_scatter(win_v, [ldc], e16, mask=valid)
        return 0

    lax.fori_loop(0, EPS // L, body, 0)
    pltpu.sync_copy(win_v, sh_v.at[pl.ds(sid * HALF, HALF)])
    pltpu.make_async_copy(src_hbm.at[pl.ds(0, EPS)], src_v, sem_b).wait()
    pltpu.sync_copy(src_v, src_sh.at[pl.ds(base_e, EPS)])
    plsc.subcore_barrier()

    for t in range(NS):
        pltpu.async_copy(sh_v.at[pl.ds(t * HALF + sid * NPT, NPT)],
                         tabs_v.at[pl.ds(t * NPT, NPT)], sem_a)
    for t in range(NS):
        pltpu.make_async_copy(sh_v.at[pl.ds(sid * NPT, NPT)],
                              tabs_v.at[pl.ds(0, NPT)], sem_a).wait()

    def merge_body(i, _):
        m = tabs_v[pl.ds(i * L, L)]
        for t in range(1, NS):
            m = jnp.maximum(m, tabs_v[pl.ds(t * NPT + i * L, L)])
        tabs_v[pl.ds(i * L, L)] = jnp.maximum(m, 0)
        tabs_v[pl.ds(NPT + i * L, L)] = (m >> 31)
        return 0

    lax.fori_loop(0, NPT // L, merge_body, 0)

    for c in range(NPT // GC):
        pltpu.async_copy(src_sh.at[tabs_v.at[pl.ds(c * GC, GC)]],
                         sidx_v.at[pl.ds(c * GC, GC)], sem_a)
    for c in range(NPT // GC):
        pltpu.make_async_copy(src_sh.at[tabs_v.at[pl.ds(0, GC)]],
                              sidx_v.at[pl.ds(0, GC)], sem_a).wait()

    def fix_body(i, _):
        miss = tabs_v[pl.ds(NPT + i * L, L)]
        s16 = sidx_v[pl.ds(i * L, L)]
        own = lo + sid * NPT + i * L + lane
        sidx_v[pl.ds(i * L, L)] = jnp.where(miss != 0, own, s16)
        return 0

    lax.fori_loop(0, NPT // L, fix_body, 0)

    for c in range(NPT // GC):
        pltpu.async_copy(x_hbm.at[sidx_v.at[pl.ds(c * GC, GC)]],
                         xrows_v.at[pl.ds(c * GC, GC)], sem_b)
    for c in range(NPT // GC):
        pltpu.make_async_copy(x_hbm.at[sidx_v.at[pl.ds(0, GC)]],
                              xrows_v.at[pl.ds(0, GC)], sem_b).wait()
    pltpu.sync_copy(xrows_v, xs_hbm.at[pl.ds(lo + sid * NPT, NPT)])


def _dense_body(x_ref, xs_ref, wq, bq, wk, bk, wv, bv, wr, br, gma, bta, ws,
                bs, wsc, out_ref):
    xs = xs_ref[...]
    xx = x_ref[...]
    q = jnp.dot(xs, wq[...], preferred_element_type=jnp.float32) + bq[...]
    k = jnp.dot(xx, wk[...], preferred_element_type=jnp.float32) + bk[...]
    v = jnp.dot(xx, wv[...], preferred_element_type=jnp.float32) + bv[...]
    scores = (q * k) * (1.0 / 8.0)
    scores = scores - jnp.max(scores, axis=-1, keepdims=True)
    ex = jnp.exp(scores)
    attn = ex / jnp.sum(ex, axis=-1, keepdims=True)
    attn_out = attn * v
    mu = jnp.mean(attn_out, axis=-1, keepdims=True)
    ctr = attn_out - mu
    var = jnp.mean(ctr * ctr, axis=-1, keepdims=True)
    xn = ctr * lax.rsqrt(var + 1e-5) * gma[...] + bta[...]
    h = attn_out + jnp.maximum(
        jnp.dot(xn, wr[...], preferred_element_type=jnp.float32) + br[...], 0.0)
    gate = jnp.dot(h, ws[...], preferred_element_type=jnp.float32) + bs[...] \
        + attn_out
    out_ref[...] = xs - xx + jnp.dot(gate, wsc[...],
                                     preferred_element_type=jnp.float32)


def _dense(x_pad, xs, hp, w_scale):
    blk = 2560
    grid = NPAD // blk

    def row_spec(dim):
        return pl.BlockSpec((blk, dim), lambda i: (i, 0))

    def full_spec(a):
        return pl.BlockSpec(a.shape, lambda i: (0,) * a.ndim)

    weights = [hp['Wq'], hp['bq'].reshape(1, G), hp['Wk'], hp['bk'].reshape(1, G),
               hp['Wv'], hp['bv'].reshape(1, G), hp['res'][0]['W'],
               hp['res'][0]['b'].reshape(1, G), hp['res'][0]['gamma'].reshape(1, G),
               hp['res'][0]['beta'].reshape(1, G), hp['Ws'],
               hp['bs'].reshape(1, G), w_scale]
    return pl.pallas_call(
        _dense_body,
        out_shape=jax.ShapeDtypeStruct((NPAD, D), jnp.float32),
        grid=(grid,),
        in_specs=[row_spec(D), row_spec(D)] + [full_spec(w) for w in weights],
        out_specs=row_spec(D),
    )(x_pad, xs, *weights)


@functools.partial(
    pl.kernel,
    out_type=jax.ShapeDtypeStruct((N_EDGES, D), jnp.float32),
    mesh=_mesh,
    scratch_types=[
        pltpu.VMEM((EPT,), jnp.int32),
        pltpu.VMEM((GD, D), jnp.float32),
        pltpu.VMEM((GD, D), jnp.float32),
        pltpu.VMEM_SHARED((NPAD, D), jnp.float32),
        pltpu.SemaphoreType.DMA,
        pltpu.SemaphoreType.DMA,
        pltpu.SemaphoreType.DMA,
        pltpu.SemaphoreType.DMA,
        pltpu.SemaphoreType.DMA,
    ],
)
def _edge_gather(h_hbm, dst_hbm, out_hbm, dst_v, rows_a, rows_b, h_sh,
                 sem_l, sem_ga, sem_gb, sem_sa, sem_sb):
    wid = _wid()
    sid = lax.axis_index("s")
    base_e = wid * EPT
    rows_per_sub = NPAD // NS
    pltpu.async_copy(dst_hbm.at[pl.ds(base_e, EPT)], dst_v, sem_l)
    pltpu.sync_copy(h_hbm.at[pl.ds(sid * rows_per_sub, rows_per_sub)],
                    h_sh.at[pl.ds(sid * rows_per_sub, rows_per_sub)])
    pltpu.make_async_copy(dst_hbm.at[pl.ds(base_e, EPT)], dst_v, sem_l).wait()
    plsc.subcore_barrier()

    def g_start(c, buf, sem):
        pltpu.async_copy(h_sh.at[dst_v.at[pl.ds(c * GD, GD)]], buf, sem)

    def g_wait(buf, sem):
        pltpu.make_async_copy(h_sh.at[dst_v.at[pl.ds(0, GD)]], buf, sem).wait()

    def s_start(c, buf, sem):
        pltpu.async_copy(buf, out_hbm.at[pl.ds(base_e + c * GD, GD)], sem)

    def s_wait(buf, sem):
        pltpu.make_async_copy(buf, out_hbm.at[pl.ds(base_e, GD)], sem).wait()

    g_start(0, rows_a, sem_ga)

    def body(i, _):
        c = 2 * i

        @pl.when(i > 0)
        def _():
            s_wait(rows_b, sem_sb)

        g_start(c + 1, rows_b, sem_gb)
        g_wait(rows_a, sem_ga)
        s_start(c, rows_a, sem_sa)
        s_wait(rows_a, sem_sa)
        g_start(c + 2, rows_a, sem_ga)
        g_wait(rows_b, sem_gb)
        s_start(c + 1, rows_b, sem_sb)
        return 0

    n_pairs = (EPT // GD) // 2
    lax.fori_loop(0, n_pairs, body, 0)
    s_wait(rows_b, sem_sb)
    g_wait(rows_a, sem_ga)
    s_start(EPT // GD - 1, rows_a, sem_sa)
    s_wait(rows_a, sem_sa)


def kernel(x, edge_index, params):
    src, dst = _split(edge_index)
    hp = params['heads'][0]
    x_pad = jnp.concatenate(
        [x, jnp.zeros((NPAD - N_NODES, D), jnp.float32)], axis=0)

    xs = _winners_gather(dst, src, x_pad)
    h_node = _dense(x_pad, xs, hp, params['W_scale'])
    return _edge_gather(h_node, dst)
